# Initial kernel scaffold; baseline (speedup 1.0000x reference)
import functools

import jax
import jax.numpy as jnp
from jax import lax
from jax.experimental import pallas as pl
from jax.experimental.pallas import tpu as pltpu
from jax.experimental.pallas import tpu_sc as plsc

N = 10000
E = 320000
D = 128

NC = 2
NS = 16
NW = NC * NS

EPW = E // NW
B = 80
NB = EPW // B
RPS = N // NS
ZR = 125

_mesh = plsc.VectorSubcoreMesh(core_axis_name="c", subcore_axis_name="s")


@functools.partial(
    pl.kernel,
    out_type=[
        jax.ShapeDtypeStruct((NC * N, D), jnp.float32),
        jax.ShapeDtypeStruct((NW, N), jnp.float32),
    ],
    mesh=_mesh,
    scratch_types=[
        pltpu.VMEM((B,), jnp.int32),
        pltpu.VMEM((B,), jnp.int32),
        pltpu.VMEM((B, D), jnp.float32),
        pltpu.VMEM((N,), jnp.float32),
        pltpu.VMEM((ZR, D), jnp.float32),
        pltpu.VMEM_SHARED((N, D), jnp.float32),
        pltpu.SemaphoreType.DMA,
    ],
)
def _phase1(x_hbm, src_hbm, dst_hbm, agg_out, deg_out,
            srcb, dstb, rowb, degb, zbuf, agg_sh, sem):
    cid = lax.axis_index("c")
    sid = lax.axis_index("s")
    wid = sid * NC + cid

    zeros16 = jnp.zeros((16,), jnp.float32)
    ones16 = jnp.full((16,), 1.0, jnp.float32)

    def zrow(r, carry):
        for c8 in range(D // 16):
            zbuf[r, pl.ds(c8 * 16, 16)] = zeros16
        return carry
    lax.fori_loop(0, ZR, zrow, 0)

    def zdeg(i, carry):
        degb[pl.ds(i * 16, 16)] = zeros16
        return carry
    lax.fori_loop(0, N // 16, zdeg, 0)

    for k in range(RPS // ZR):
        pltpu.sync_copy(zbuf, agg_sh.at[pl.ds(sid * RPS + k * ZR, ZR)])
    plsc.subcore_barrier()

    base0 = wid * EPW

    def body(g, carry):
        base = base0 + g * B
        pltpu.sync_copy(src_hbm.at[pl.ds(base, B)], srcb)
        pltpu.sync_copy(dst_hbm.at[pl.ds(base, B)], dstb)
        pltpu.async_copy(x_hbm.at[srcb], rowb, sem).wait()
        pltpu.sync_copy(rowb, agg_sh.at[dstb], add=True)
        for t in range(B // 16):
            dv = dstb[pl.ds(t * 16, 16)]
            plsc.addupdate_scatter(degb, [dv], ones16)
        return carry
    lax.fori_loop(0, NB, body, 0)

    plsc.subcore_barrier()
    pltpu.sync_copy(agg_sh.at[pl.ds(sid * RPS, RPS)],
                    agg_out.at[pl.ds(cid * N + sid * RPS, RPS)])
    pltpu.sync_copy(degb, deg_out.at[wid])


@functools.partial(
    pl.kernel,
    out_type=jax.ShapeDtypeStruct((E,), jnp.float32),
    mesh=_mesh,
    scratch_types=[
        pltpu.VMEM((B,), jnp.int32),
        pltpu.VMEM((B,), jnp.int32),
        pltpu.VMEM((B, D), jnp.float32),
        pltpu.VMEM((B, D), jnp.float32),
        pltpu.VMEM((16, 17), jnp.float32),
        pltpu.VMEM((B,), jnp.float32),
        pltpu.SemaphoreType.DMA,
    ],
)
def _phase3(ne_hbm, src_hbm, dst_hbm, pred_out,
            srcb, dstb, srows, drows, trans, dots, sem):
    cid = lax.axis_index("c")
    sid = lax.axis_index("s")
    wid = sid * NC + cid
    base0 = wid * EPW

    row_iota = lax.iota(jnp.int32, 16)

    def body(g, carry):
        base = base0 + g * B
        pltpu.sync_copy(src_hbm.at[pl.ds(base, B)], srcb)
        pltpu.sync_copy(dst_hbm.at[pl.ds(base, B)], dstb)
        cp1 = pltpu.async_copy(ne_hbm.at[srcb], srows, sem)
        cp2 = pltpu.async_copy(ne_hbm.at[dstb], drows, sem)
        cp1.wait()
        cp2.wait()
        for e0 in range(0, B, 16):
            def edot(i, carry2):
                e = e0 + i
                acc = srows[e, pl.ds(0, 16)] * drows[e, pl.ds(0, 16)]
                for c in range(1, D // 16):
                    acc = acc + srows[e, pl.ds(c * 16, 16)] * drows[e, pl.ds(c * 16, 16)]
                trans[i, pl.ds(0, 16)] = acc
                return carry2
            lax.fori_loop(0, 16, edot, 0)
            tot = jnp.zeros((16,), jnp.float32)
            for j in range(16):
                col = plsc.load_gather(trans, [row_iota, jnp.full((16,), j, jnp.int32)])
                tot = tot + col
            dots[pl.ds(e0, 16)] = tot
        pltpu.sync_copy(dots, pred_out.at[pl.ds(base, B)])
        return carry
    lax.fori_loop(0, NB, body, 0)


_R2 = 2000


def _phase2_body(aggp_ref, degp_ref, w1_ref, b1_ref, w2_ref, b2_ref, out_ref):
    agg = aggp_ref[0] + aggp_ref[1]
    deg = jnp.sum(degp_ref[...], axis=0)
    deg = jnp.maximum(deg, 1.0)
    h = agg / deg[:, None]
    h = jnp.maximum(jnp.dot(h, w1_ref[...], preferred_element_type=jnp.float32)
                    + b1_ref[...], 0.0)
    out_ref[...] = (jnp.dot(h, w2_ref[...], preferred_element_type=jnp.float32)
                    + b2_ref[...])


def _phase4_body(pred_ref, y_ref, out_ref):
    p = pred_ref[...]
    y = y_ref[...]
    bce = jnp.maximum(p, 0.0) - p * y + jnp.log1p(jnp.exp(-jnp.abs(p)))
    out_ref[0, 0] = jnp.sum(bce) * (1.0 / E)


def kernel(x, edge_index, edge_label, W_gnn, b_gnn, W_pred, b_pred):
    src = edge_index[0]
    dst = edge_index[1]

    agg_p, deg_p = _phase1(x, src, dst)

    agg_p = agg_p.reshape(NC, N, D)
    node_emb = pl.pallas_call(
        _phase2_body,
        grid=(N // _R2,),
        in_specs=[
            pl.BlockSpec((NC, _R2, D), lambda i: (0, i, 0)),
            pl.BlockSpec((NW, _R2), lambda i: (0, i)),
            pl.BlockSpec((D, D), lambda i: (0, 0)),
            pl.BlockSpec((1, D), lambda i: (0, 0)),
            pl.BlockSpec((D, D), lambda i: (0, 0)),
            pl.BlockSpec((1, D), lambda i: (0, 0)),
        ],
        out_specs=pl.BlockSpec((_R2, D), lambda i: (i, 0)),
        out_shape=jax.ShapeDtypeStruct((N, D), jnp.float32),
    )(agg_p, deg_p, W_gnn, b_gnn.reshape(1, D), W_pred, b_pred.reshape(1, D))

    pred = _phase3(node_emb, src, dst)

    loss2d = pl.pallas_call(
        _phase4_body,
        in_specs=[
            pl.BlockSpec((E // D, D), lambda: (0, 0)),
            pl.BlockSpec((E // D, D), lambda: (0, 0)),
        ],
        out_specs=pl.BlockSpec((1, 1), lambda: (0, 0), memory_space=pltpu.SMEM),
        out_shape=jax.ShapeDtypeStruct((1, 1), jnp.float32),
    )(pred.reshape(E // D, D), edge_label.astype(jnp.float32).reshape(E // D, D))

    return loss2d[0, 0]


# speedup vs baseline: 4.8324x; 4.8324x over previous
import functools

import jax
import jax.numpy as jnp
from jax import lax
from jax.experimental import pallas as pl
from jax.experimental.pallas import tpu as pltpu
from jax.experimental.pallas import tpu_sc as plsc

N = 10000
E = 320000
D = 128

NC = 2
NS = 16
NW = NC * NS

EPW = E // NW
B = 80
NB = EPW // B
CH = 80
NCH = N // CH

_mesh = plsc.VectorSubcoreMesh(core_axis_name="c", subcore_axis_name="s")


@functools.partial(
    pl.kernel,
    out_type=[
        jax.ShapeDtypeStruct((NC * N, D), jnp.float32),
        jax.ShapeDtypeStruct((NW * N,), jnp.float32),
    ],
    mesh=_mesh,
    compiler_params=pltpu.CompilerParams(needs_layout_passes=False),
    scratch_types=[
        pltpu.VMEM((B,), jnp.int32),
        pltpu.VMEM((B,), jnp.int32),
        pltpu.VMEM((B, D), jnp.float32),
        pltpu.VMEM((N,), jnp.float32),
        pltpu.VMEM((CH, D), jnp.float32),
        pltpu.VMEM_SHARED((N, D), jnp.float32),
        pltpu.SemaphoreType.DMA,
    ],
)
def _phase1(x_hbm, src_hbm, dst_hbm, agg_out, deg_out,
            srcb, dstb, rowb, degb, zbuf, agg_sh, sem):
    cid = lax.axis_index("c")
    sid = lax.axis_index("s")
    wid = sid * NC + cid

    zeros16 = jnp.zeros((16,), jnp.float32)
    ones16 = jnp.full((16,), 1.0, jnp.float32)

    def zrow(r, carry):
        for c8 in range(D // 16):
            zbuf[r, pl.ds(c8 * 16, 16)] = zeros16
        return carry
    lax.fori_loop(0, CH, zrow, 0)

    def zdeg(i, carry):
        degb[pl.ds(i * 16, 16)] = zeros16
        return carry
    lax.fori_loop(0, N // 16, zdeg, 0)

    def zchunk(k, carry):
        c = sid + k * NS

        @pl.when(c < NCH)
        def _():
            pltpu.sync_copy(zbuf, agg_sh.at[pl.ds(c * CH, CH)])
        return carry
    lax.fori_loop(0, pl.cdiv(NCH, NS), zchunk, 0)
    plsc.subcore_barrier()

    base0 = wid * EPW

    def body(g, carry):
        base = base0 + g * B
        pltpu.sync_copy(src_hbm.at[pl.ds(base, B)], srcb)
        pltpu.sync_copy(dst_hbm.at[pl.ds(base, B)], dstb)
        pltpu.async_copy(x_hbm.at[srcb], rowb, sem).wait()
        pltpu.sync_copy(rowb, agg_sh.at[dstb], add=True)
        for t in range(B // 16):
            dv = dstb[pl.ds(t * 16, 16)]
            plsc.addupdate_scatter(degb, [dv], ones16)
        return carry
    lax.fori_loop(0, NB, body, 0)

    plsc.subcore_barrier()

    def wchunk(k, carry):
        c = sid + k * NS

        @pl.when(c < NCH)
        def _():
            pltpu.sync_copy(agg_sh.at[pl.ds(c * CH, CH)],
                            agg_out.at[pl.ds(cid * N + c * CH, CH)])
        return carry
    lax.fori_loop(0, pl.cdiv(NCH, NS), wchunk, 0)
    pltpu.sync_copy(degb, deg_out.at[pl.ds(wid * N, N)])


@functools.partial(
    pl.kernel,
    out_type=jax.ShapeDtypeStruct((E,), jnp.float32),
    mesh=_mesh,
    compiler_params=pltpu.CompilerParams(needs_layout_passes=False),
    scratch_types=[
        pltpu.VMEM((B,), jnp.int32),
        pltpu.VMEM((B,), jnp.int32),
        pltpu.VMEM((B, D), jnp.float32),
        pltpu.VMEM((B, D), jnp.float32),
        pltpu.VMEM((16, 17), jnp.float32),
        pltpu.VMEM((B,), jnp.float32),
        pltpu.SemaphoreType.DMA,
    ],
)
def _phase3(ne_hbm, src_hbm, dst_hbm, pred_out,
            srcb, dstb, srows, drows, trans, dots, sem):
    cid = lax.axis_index("c")
    sid = lax.axis_index("s")
    wid = sid * NC + cid
    base0 = wid * EPW

    row_iota = lax.iota(jnp.int32, 16)

    def body(g, carry):
        base = base0 + g * B
        pltpu.sync_copy(src_hbm.at[pl.ds(base, B)], srcb)
        pltpu.sync_copy(dst_hbm.at[pl.ds(base, B)], dstb)
        cp1 = pltpu.async_copy(ne_hbm.at[srcb], srows, sem)
        cp2 = pltpu.async_copy(ne_hbm.at[dstb], drows, sem)
        cp1.wait()
        cp2.wait()
        for e0 in range(0, B, 16):
            def edot(i, carry2):
                e = e0 + i
                acc = srows[e, pl.ds(0, 16)] * drows[e, pl.ds(0, 16)]
                for c in range(1, D // 16):
                    acc = acc + srows[e, pl.ds(c * 16, 16)] * drows[e, pl.ds(c * 16, 16)]
                trans[i, pl.ds(0, 16)] = acc
                return carry2
            lax.fori_loop(0, 16, edot, 0)
            tot = jnp.zeros((16,), jnp.float32)
            for j in range(16):
                col = plsc.load_gather(trans, [row_iota, jnp.full((16,), j, jnp.int32)])
                tot = tot + col
            dots[pl.ds(e0, 16)] = tot
        pltpu.sync_copy(dots, pred_out.at[pl.ds(base, B)])
        return carry
    lax.fori_loop(0, NB, body, 0)


def _phase2_body(aggp_ref, degp_ref, w1_ref, b1_ref, w2_ref, b2_ref, out_ref):
    agg = aggp_ref[0] + aggp_ref[1]
    deg = jnp.sum(degp_ref[...], axis=0)
    deg = jnp.maximum(deg, 1.0)
    h = agg / deg[:, None]
    h = jnp.maximum(jnp.dot(h, w1_ref[...], preferred_element_type=jnp.float32)
                    + b1_ref[...], 0.0)
    out_ref[...] = (jnp.dot(h, w2_ref[...], preferred_element_type=jnp.float32)
                    + b2_ref[...])


def _phase4_body(pred_ref, y_ref, out_ref):
    p = pred_ref[...]
    y = y_ref[...]
    bce = jnp.maximum(p, 0.0) - p * y + jnp.log1p(jnp.exp(-jnp.abs(p)))
    out_ref[0, 0] = jnp.sum(bce) * (1.0 / E)


def kernel(x, edge_index, edge_label, W_gnn, b_gnn, W_pred, b_pred):
    src = edge_index[0]
    dst = edge_index[1]

    agg_p, deg_p = _phase1(x, src, dst)

    agg_p = agg_p.reshape(NC, N, D)
    deg_p = deg_p.reshape(NW, N)
    node_emb = pl.pallas_call(
        _phase2_body,
        out_shape=jax.ShapeDtypeStruct((N, D), jnp.float32),
    )(agg_p, deg_p, W_gnn, b_gnn.reshape(1, D), W_pred, b_pred.reshape(1, D))

    pred = _phase3(node_emb, src, dst)

    loss2d = pl.pallas_call(
        _phase4_body,
        in_specs=[
            pl.BlockSpec((E // D, D), lambda: (0, 0)),
            pl.BlockSpec((E // D, D), lambda: (0, 0)),
        ],
        out_specs=pl.BlockSpec((1, 1), lambda: (0, 0), memory_space=pltpu.SMEM),
        out_shape=jax.ShapeDtypeStruct((1, 1), jnp.float32),
    )(pred.reshape(E // D, D), edge_label.astype(jnp.float32).reshape(E // D, D))

    return loss2d[0, 0]


# speedup vs baseline: 11.1773x; 2.3130x over previous
import functools

import jax
import jax.numpy as jnp
from jax import lax
from jax.experimental import pallas as pl
from jax.experimental.pallas import tpu as pltpu
from jax.experimental.pallas import tpu_sc as plsc

N = 10000
E = 320000
D = 128

NC = 2
NS = 16
NW = NC * NS

EPW = E // NW
B = 80
NB = EPW // B
CH = 80
NCH = N // CH
ZCH = 8
NZCH = N // ZCH

_mesh = plsc.VectorSubcoreMesh(core_axis_name="c", subcore_axis_name="s")
_params = pltpu.CompilerParams(needs_layout_passes=False)


@functools.partial(
    pl.kernel,
    out_type=[
        jax.ShapeDtypeStruct((NC * N, D), jnp.float32),
        jax.ShapeDtypeStruct((NW * N,), jnp.float32),
    ],
    mesh=_mesh,
    compiler_params=_params,
    scratch_types=[
        pltpu.VMEM((B,), jnp.int32),
        pltpu.VMEM((B,), jnp.int32),
        pltpu.VMEM((NB, B), jnp.int32),
        pltpu.VMEM((B, D), jnp.float32),
        pltpu.VMEM((B, D), jnp.float32),
        pltpu.VMEM((N,), jnp.float32),
        pltpu.VMEM((ZCH, D), jnp.float32),
        pltpu.VMEM_SHARED((N, D), jnp.float32),
        pltpu.SemaphoreType.DMA,
        pltpu.SemaphoreType.DMA,
        pltpu.SemaphoreType.DMA,
        pltpu.SemaphoreType.DMA,
    ],
)
def _phase1(x_hbm, src_hbm, dst3_hbm, agg_out, deg_out,
            sib0, sib1, dstall, rowb0, rowb1, degb, zbuf, agg_sh,
            g0, g1, i0, i1):
    cid = lax.axis_index("c")
    sid = lax.axis_index("s")
    wid = sid * NC + cid

    zeros16 = jnp.zeros((16,), jnp.float32)
    ones16 = jnp.full((16,), 1.0, jnp.float32)

    pltpu.sync_copy(dst3_hbm.at[wid], dstall)

    def zrow(r, carry):
        for c8 in range(D // 16):
            zbuf[r, pl.ds(c8 * 16, 16)] = zeros16
        return carry
    lax.fori_loop(0, ZCH, zrow, 0)

    def zdeg(i, carry):
        degb[pl.ds(i * 16, 16)] = zeros16
        return carry
    lax.fori_loop(0, N // 16, zdeg, 0)

    def zchunk(k, carry):
        c = sid + k * NS

        @pl.when(c < NZCH)
        def _():
            pltpu.sync_copy(zbuf, agg_sh.at[pl.ds(c * ZCH, ZCH)])
        return carry
    lax.fori_loop(0, pl.cdiv(NZCH, NS), zchunk, 0)
    plsc.subcore_barrier()

    base0 = wid * EPW

    def idx_issue(g, buf, sem):
        pltpu.async_copy(src_hbm.at[pl.ds(base0 + g * B, B)], buf, sem)

    def idx_wait(g, buf, sem):
        pltpu.make_async_copy(src_hbm.at[pl.ds(base0 + g * B, B)], buf, sem).wait()

    def gather(buf, ibuf, sem):
        pltpu.async_copy(x_hbm.at[ibuf], buf, sem)

    def gather_wait(buf, ibuf, sem):
        pltpu.make_async_copy(x_hbm.at[ibuf], buf, sem).wait()

    pltpu.sync_copy(src_hbm.at[pl.ds(base0, B)], sib0)
    gather(rowb0, sib0, g0)
    idx_issue(1, sib1, i1)

    def body(g, carry):
        even = lax.rem(g, 2) == 0
        odd = lax.rem(g, 2) == 1

        @pl.when(g + 1 < NB)
        def _():
            @pl.when(even)
            def _():
                idx_wait(g + 1, sib1, i1)
                gather(rowb1, sib1, g1)

            @pl.when(odd)
            def _():
                idx_wait(g + 1, sib0, i0)
                gather(rowb0, sib0, g0)

        @pl.when(even)
        def _():
            gather_wait(rowb0, sib0, g0)

        @pl.when(odd)
        def _():
            gather_wait(rowb1, sib1, g1)

        @pl.when(g + 2 < NB)
        def _():
            @pl.when(even)
            def _():
                idx_issue(g + 2, sib0, i0)

            @pl.when(odd)
            def _():
                idx_issue(g + 2, sib1, i1)

        for t in range(B // 16):
            dv = dstall[g, pl.ds(t * 16, 16)]
            plsc.addupdate_scatter(degb, [dv], ones16)

        @pl.when(even)
        def _():
            pltpu.sync_copy(rowb0, agg_sh.at[dstall.at[g]], add=True)

        @pl.when(odd)
        def _():
            pltpu.sync_copy(rowb1, agg_sh.at[dstall.at[g]], add=True)
        return carry
    lax.fori_loop(0, NB, body, 0)

    plsc.subcore_barrier()

    def wchunk(k, carry):
        c = sid + k * NS

        @pl.when(c < NCH)
        def _():
            pltpu.sync_copy(agg_sh.at[pl.ds(c * CH, CH)],
                            agg_out.at[pl.ds(cid * N + c * CH, CH)])
        return carry
    lax.fori_loop(0, pl.cdiv(NCH, NS), wchunk, 0)
    pltpu.sync_copy(degb, deg_out.at[pl.ds(wid * N, N)])


@functools.partial(
    pl.kernel,
    out_type=jax.ShapeDtypeStruct((E,), jnp.float32),
    mesh=_mesh,
    compiler_params=_params,
    scratch_types=[
        pltpu.VMEM((EPW,), jnp.int32),
        pltpu.VMEM((EPW,), jnp.int32),
        pltpu.VMEM((B, D), jnp.float32),
        pltpu.VMEM((B, D), jnp.float32),
        pltpu.VMEM((B, D), jnp.float32),
        pltpu.VMEM((B, D), jnp.float32),
        pltpu.VMEM((16, 17), jnp.float32),
        pltpu.VMEM((EPW,), jnp.float32),
        pltpu.SemaphoreType.DMA,
        pltpu.SemaphoreType.DMA,
        pltpu.SemaphoreType.DMA,
        pltpu.SemaphoreType.DMA,
    ],
)
def _phase3(ne_hbm, src_hbm, dst_hbm, pred_out,
            srcall, dstall, sr0, dr0, sr1, dr1, trans, dots, s0, d0, s1, d1):
    cid = lax.axis_index("c")
    sid = lax.axis_index("s")
    wid = sid * NC + cid

    row_iota = lax.iota(jnp.int32, 16)

    pltpu.sync_copy(src_hbm.at[pl.ds(wid * EPW, EPW)], srcall)
    pltpu.sync_copy(dst_hbm.at[pl.ds(wid * EPW, EPW)], dstall)

    def issue(g, sbuf, dbuf, ssem, dsem):
        pltpu.async_copy(ne_hbm.at[srcall.at[pl.ds(g * B, B)]], sbuf, ssem)
        pltpu.async_copy(ne_hbm.at[dstall.at[pl.ds(g * B, B)]], dbuf, dsem)

    def wait(g, sbuf, dbuf, ssem, dsem):
        pltpu.make_async_copy(ne_hbm.at[srcall.at[pl.ds(g * B, B)]], sbuf, ssem).wait()
        pltpu.make_async_copy(ne_hbm.at[dstall.at[pl.ds(g * B, B)]], dbuf, dsem).wait()

    def compute(g, sbuf, dbuf):
        for e0 in range(0, B, 16):
            def edot(i, carry2):
                e = e0 + i
                acc = sbuf[e, pl.ds(0, 16)] * dbuf[e, pl.ds(0, 16)]
                for c in range(1, D // 16):
                    acc = acc + sbuf[e, pl.ds(c * 16, 16)] * dbuf[e, pl.ds(c * 16, 16)]
                trans[i, pl.ds(0, 16)] = acc
                return carry2
            lax.fori_loop(0, 16, edot, 0)
            tot = jnp.zeros((16,), jnp.float32)
            for j in range(16):
                col = plsc.load_gather(trans, [row_iota, jnp.full((16,), j, jnp.int32)])
                tot = tot + col
            dots[pl.ds(g * B + e0, 16)] = tot

    issue(0, sr0, dr0, s0, d0)

    def body(g, carry):
        even = lax.rem(g, 2) == 0
        odd = lax.rem(g, 2) == 1

        @pl.when(g + 1 < NB)
        def _():
            @pl.when(even)
            def _():
                issue(g + 1, sr1, dr1, s1, d1)

            @pl.when(odd)
            def _():
                issue(g + 1, sr0, dr0, s0, d0)

        @pl.when(even)
        def _():
            wait(g, sr0, dr0, s0, d0)
            compute(g, sr0, dr0)

        @pl.when(odd)
        def _():
            wait(g, sr1, dr1, s1, d1)
            compute(g, sr1, dr1)
        return carry
    lax.fori_loop(0, NB, body, 0)

    pltpu.sync_copy(dots, pred_out.at[pl.ds(wid * EPW, EPW)])


def _phase2_body(aggp_ref, degp_ref, w1_ref, b1_ref, w2_ref, b2_ref, out_ref):
    agg = aggp_ref[0] + aggp_ref[1]
    deg = jnp.sum(degp_ref[...], axis=0)
    deg = jnp.maximum(deg, 1.0)
    h = agg / deg[:, None]
    h = jnp.maximum(jnp.dot(h, w1_ref[...], preferred_element_type=jnp.float32)
                    + b1_ref[...], 0.0)
    out_ref[...] = (jnp.dot(h, w2_ref[...], preferred_element_type=jnp.float32)
                    + b2_ref[...])


def _phase4_body(pred_ref, y_ref, out_ref):
    p = pred_ref[...]
    y = y_ref[...]
    bce = jnp.maximum(p, 0.0) - p * y + jnp.log1p(jnp.exp(-jnp.abs(p)))
    out_ref[0, 0] = jnp.sum(bce) * (1.0 / E)


def kernel(x, edge_index, edge_label, W_gnn, b_gnn, W_pred, b_pred):
    src = edge_index[0]
    dst = edge_index[1]

    agg_p, deg_p = _phase1(x, src, dst.reshape(NW, NB, B))

    agg_p = agg_p.reshape(NC, N, D)
    deg_p = deg_p.reshape(NW, N)
    node_emb = pl.pallas_call(
        _phase2_body,
        out_shape=jax.ShapeDtypeStruct((N, D), jnp.float32),
    )(agg_p, deg_p, W_gnn, b_gnn.reshape(1, D), W_pred, b_pred.reshape(1, D))

    pred = _phase3(node_emb, src, dst)

    loss2d = pl.pallas_call(
        _phase4_body,
        in_specs=[
            pl.BlockSpec((E // D, D), lambda: (0, 0)),
            pl.BlockSpec((E // D, D), lambda: (0, 0)),
        ],
        out_specs=pl.BlockSpec((1, 1), lambda: (0, 0), memory_space=pltpu.SMEM),
        out_shape=jax.ShapeDtypeStruct((1, 1), jnp.float32),
    )(pred.reshape(E // D, D), edge_label.astype(jnp.float32).reshape(E // D, D))

    return loss2d[0, 0]
